# Initial kernel scaffold; baseline (speedup 1.0000x reference)
import jax
import jax.numpy as jnp
from jax.experimental import pallas as pl


def kernel(x, edge_index, W1, b1, W2, b2, Wm, bm):
    raise NotImplementedError("write your pallas kernel here")


# speedup vs baseline: 12.4434x; 12.4434x over previous
import functools

import jax
import jax.numpy as jnp
from jax import lax
from jax.experimental import pallas as pl
from jax.experimental.pallas import tpu as pltpu
from jax.experimental.pallas import tpu_sc as plsc

N = 10000
E = 320000
D = 128
H = 32
C = 2

NP = 10240
NC = 2
NS = 16
NW = NC * NS
EPW = E // NW
B = 128
CH = -(-EPW // B)
EPWP = CH * B
PAD = NP - 1
RPT = NP // NS
NR = 6

_mesh = plsc.VectorSubcoreMesh(
    core_axis_name="c", subcore_axis_name="s", num_cores=NC, num_subcores=NS)


def _deg_body(srcp, dstp, ones2, zeros8, out, src_v, dst_v, ones_v, deg_sh, sems):
    c = lax.axis_index("c")
    s = lax.axis_index("s")
    wid = c * NS + s
    pltpu.sync_copy(srcp.at[wid], src_v)
    pltpu.sync_copy(dstp.at[wid], dst_v)
    pltpu.sync_copy(ones2, ones_v)
    pltpu.sync_copy(zeros8.at[0, pl.ds(s * RPT, RPT)], deg_sh.at[0, pl.ds(s * RPT, RPT)])
    pltpu.sync_copy(zeros8.at[1, pl.ds(s * RPT, RPT)], deg_sh.at[1, pl.ds(s * RPT, RPT)])
    plsc.subcore_barrier()

    def body(j, carry):
        cp0 = pltpu.async_copy(ones_v.at[0], deg_sh.at[0].at[src_v.at[j]], sems.at[0], add=True)
        cp1 = pltpu.async_copy(ones_v.at[1], deg_sh.at[1].at[dst_v.at[j]], sems.at[1], add=True)
        cp0.wait()
        cp1.wait()
        return carry

    lax.fori_loop(0, CH, body, 0)
    plsc.subcore_barrier()
    pltpu.sync_copy(deg_sh.at[0, pl.ds(s * RPT, RPT)], out.at[c, 0, pl.ds(s * RPT, RPT)])
    pltpu.sync_copy(deg_sh.at[1, pl.ds(s * RPT, RPT)], out.at[c, 1, pl.ds(s * RPT, RPT)])


_deg = functools.partial(
    pl.kernel,
    out_type=jax.ShapeDtypeStruct((NC, 2, NP, 8), jnp.float32),
    mesh=_mesh,
    compiler_params=pltpu.CompilerParams(use_tc_tiling_on_sc=False),
    scratch_types=[
        pltpu.VMEM((CH, B), jnp.int32),
        pltpu.VMEM((CH, B), jnp.int32),
        pltpu.VMEM((2, B, 8), jnp.float32),
        pltpu.VMEM_SHARED((2, NP, 8), jnp.float32),
        pltpu.SemaphoreType.DMA((2,)),
    ],
)(_deg_body)


def _conv_body(h, srcp, dstp, out, src_v, dst_v, rows_v, acc_v, agg_sh,
               gsems, ssems):
    c = lax.axis_index("c")
    s = lax.axis_index("s")
    wid = c * NS + s
    pltpu.sync_copy(srcp.at[wid], src_v)
    pltpu.sync_copy(dstp.at[wid], dst_v)

    def zero(q, carry):
        acc_v[lax.rem(q, 2), lax.div(q, 2 * (H // 16)),
              pl.ds(lax.rem(lax.div(q, 2), H // 16) * 16, 16)] = jnp.zeros(
                  (16,), jnp.float32)
        return carry

    lax.fori_loop(0, 2 * RPT * (H // 16), zero, 0)
    pltpu.sync_copy(acc_v.at[0], agg_sh.at[0, pl.ds(s * RPT, RPT)])
    pltpu.sync_copy(acc_v.at[1], agg_sh.at[1, pl.ds(s * RPT, RPT)])
    plsc.subcore_barrier()

    def g_cp(j):
        return pltpu.make_async_copy(
            h.at[src_v.at[j]], rows_v.at[lax.rem(j, NR)], gsems.at[lax.rem(j, NR)])

    def s_cp(j):
        p = lax.rem(j, 2)
        return pltpu.make_async_copy(
            rows_v.at[lax.rem(j, NR)], agg_sh.at[p].at[dst_v.at[j]], ssems.at[p])

    g_cp(0).start()
    g_cp(1).start()
    g_cp(2).start()

    def body(j, carry):
        g_cp(j).wait()

        @pl.when(j >= 2)
        def _():
            s_cp(j - 2).wait()

        s_cp(j).start(add=True)

        @pl.when(j + 3 < CH)
        def _():
            g_cp(j + 3).start()

        return carry

    lax.fori_loop(0, CH, body, 0)
    s_cp(CH - 2).wait()
    s_cp(CH - 1).wait()
    plsc.subcore_barrier()

    pltpu.sync_copy(agg_sh.at[0, pl.ds(s * RPT, RPT)], acc_v.at[0])
    pltpu.sync_copy(agg_sh.at[1, pl.ds(s * RPT, RPT)], acc_v.at[1])

    def red(q, carry):
        i = lax.div(q, H // 16)
        sl = pl.ds(lax.rem(q, H // 16) * 16, 16)
        acc_v[0, i, sl] = acc_v[0, i, sl] + acc_v[1, i, sl]
        return carry

    lax.fori_loop(0, RPT * (H // 16), red, 0)
    pltpu.sync_copy(acc_v.at[0], out.at[c, pl.ds(s * RPT, RPT)])


_conv = functools.partial(
    pl.kernel,
    out_type=jax.ShapeDtypeStruct((NC, NP, H), jnp.float32),
    mesh=_mesh,
    compiler_params=pltpu.CompilerParams(use_tc_tiling_on_sc=False),
    scratch_types=[
        pltpu.VMEM((CH, B), jnp.int32),
        pltpu.VMEM((CH, B), jnp.int32),
        pltpu.VMEM((NR, B, H), jnp.float32),
        pltpu.VMEM((2, RPT, H), jnp.float32),
        pltpu.VMEM_SHARED((2, NP, H), jnp.float32),
        pltpu.SemaphoreType.DMA((NR,)),
        pltpu.SemaphoreType.DMA((2,)),
    ],
)(_conv_body)


_BLK = 1000
_GRID = N // _BLK


def _norms(degp_ref):
    deg = degp_ref[0] + degp_ref[1]
    do_ = deg[0, :, 0:1]
    di = deg[1, :, 1:2]
    ns = jnp.where(do_ > 0, lax.rsqrt(jnp.maximum(do_, 1.0)), 0.0)
    nd = jnp.where(di > 0, lax.rsqrt(jnp.maximum(di, 1.0)), 0.0)
    return ns, nd


def _tcA_body(x_ref, w_ref, degp_ref, o_ref):
    ns, _ = _norms(degp_ref)
    o_ref[...] = jnp.dot(x_ref[...], w_ref[...],
                         preferred_element_type=jnp.float32) * ns


def _tcA(x, W1, degp):
    return pl.pallas_call(
        _tcA_body,
        grid=(_GRID,),
        in_specs=[
            pl.BlockSpec((_BLK, D), lambda i: (i, 0)),
            pl.BlockSpec((D, H), lambda i: (0, 0)),
            pl.BlockSpec((NC, 2, _BLK, 8), lambda i: (0, 0, i, 0)),
        ],
        out_specs=pl.BlockSpec((_BLK, H), lambda i: (i, 0)),
        out_shape=jax.ShapeDtypeStruct((NP, H), jnp.float32),
    )(x, W1, degp)


def _tcB_body(aggp_ref, degp_ref, b_ref, w_ref, o_ref):
    ns, nd = _norms(degp_ref)
    agg = aggp_ref[0] + aggp_ref[1]
    h = jax.nn.relu(agg * nd + b_ref[...])
    o_ref[...] = jnp.dot(h, w_ref[...], preferred_element_type=jnp.float32) * ns


def _tcB(aggp, degp, b1, W2):
    return pl.pallas_call(
        _tcB_body,
        grid=(_GRID,),
        in_specs=[
            pl.BlockSpec((NC, _BLK, H), lambda i: (0, i, 0)),
            pl.BlockSpec((NC, 2, _BLK, 8), lambda i: (0, 0, i, 0)),
            pl.BlockSpec((1, H), lambda i: (0, 0)),
            pl.BlockSpec((H, H), lambda i: (0, 0)),
        ],
        out_specs=pl.BlockSpec((_BLK, H), lambda i: (i, 0)),
        out_shape=jax.ShapeDtypeStruct((NP, H), jnp.float32),
    )(aggp, degp, b1.reshape(1, H), W2)


def _tcC_body(aggp_ref, degp_ref, b_ref, w_ref, bm_ref, o_ref):
    _, nd = _norms(degp_ref)
    agg = aggp_ref[0] + aggp_ref[1]
    h = jax.nn.relu(agg * nd + b_ref[...])
    o_ref[...] = jnp.dot(h, w_ref[...],
                         preferred_element_type=jnp.float32) + bm_ref[...]


def _tcC(aggp, degp, b2, Wm, bm):
    return pl.pallas_call(
        _tcC_body,
        grid=(_GRID,),
        in_specs=[
            pl.BlockSpec((NC, _BLK, H), lambda i: (0, i, 0)),
            pl.BlockSpec((NC, 2, _BLK, 8), lambda i: (0, 0, i, 0)),
            pl.BlockSpec((1, H), lambda i: (0, 0)),
            pl.BlockSpec((H, C), lambda i: (0, 0)),
            pl.BlockSpec((1, C), lambda i: (0, 0)),
        ],
        out_specs=pl.BlockSpec((_BLK, C), lambda i: (i, 0)),
        out_shape=jax.ShapeDtypeStruct((N, C), jnp.float32),
    )(aggp, degp, b2.reshape(1, H), Wm, bm.reshape(1, C))


def kernel(x, edge_index, W1, b1, W2, b2, Wm, bm):
    src = edge_index[0].reshape(NW, EPW)
    dst = edge_index[1].reshape(NW, EPW)
    padblk = jnp.full((NW, EPWP - EPW), PAD, jnp.int32)
    srcp = jnp.concatenate([src, padblk], axis=1).reshape(NW, CH, B)
    dstp = jnp.concatenate([dst, padblk], axis=1).reshape(NW, CH, B)

    ones2 = jnp.zeros((2, B, 8), jnp.float32).at[0, :, 0].set(1.0).at[1, :, 1].set(1.0)
    zeros8 = jnp.zeros((2, NP, 8), jnp.float32)

    degp = _deg(srcp, dstp, ones2, zeros8)
    h1a = _tcA(x, W1, degp)
    agg1 = _conv(h1a, srcp, dstp)
    h2a = _tcB(agg1, degp, b1, W2)
    agg2 = _conv(h2a, srcp, dstp)
    return _tcC(agg2, degp, b2, Wm, bm)


# speedup vs baseline: 12.7725x; 1.0264x over previous
import functools

import jax
import jax.numpy as jnp
from jax import lax
from jax.experimental import pallas as pl
from jax.experimental.pallas import tpu as pltpu
from jax.experimental.pallas import tpu_sc as plsc

N = 10000
E = 320000
D = 128
H = 32
C = 2

NP = 10240
NC = 2
NS = 16
NW = NC * NS
EPW = E // NW
B = 128
CH = -(-EPW // B)
EPWP = CH * B
PAD = NP - 1
RPT = NP // NS

_mesh = plsc.VectorSubcoreMesh(
    core_axis_name="c", subcore_axis_name="s", num_cores=NC, num_subcores=NS)


def _deg_body(srcp, dstp, ones2, zeros8, out, src_v, dst_v, ones_v, deg_sh):
    c = lax.axis_index("c")
    s = lax.axis_index("s")
    wid = c * NS + s
    pltpu.sync_copy(srcp.at[wid], src_v)
    pltpu.sync_copy(dstp.at[wid], dst_v)
    pltpu.sync_copy(ones2, ones_v)
    pltpu.sync_copy(zeros8.at[pl.ds(s * RPT, RPT)], deg_sh.at[pl.ds(s * RPT, RPT)])
    plsc.subcore_barrier()

    def body(j, carry):
        pltpu.sync_copy(ones_v.at[0], deg_sh.at[src_v.at[j]], add=True)
        pltpu.sync_copy(ones_v.at[1], deg_sh.at[dst_v.at[j]], add=True)
        return carry

    lax.fori_loop(0, CH, body, 0)
    plsc.subcore_barrier()
    pltpu.sync_copy(deg_sh.at[pl.ds(s * RPT, RPT)], out.at[c, pl.ds(s * RPT, RPT)])


_deg = functools.partial(
    pl.kernel,
    out_type=jax.ShapeDtypeStruct((NC, NP, 8), jnp.float32),
    mesh=_mesh,
    compiler_params=pltpu.CompilerParams(use_tc_tiling_on_sc=False),
    scratch_types=[
        pltpu.VMEM((CH, B), jnp.int32),
        pltpu.VMEM((CH, B), jnp.int32),
        pltpu.VMEM((2, B, 8), jnp.float32),
        pltpu.VMEM_SHARED((NP, 8), jnp.float32),
    ],
)(_deg_body)


def _conv_body(h, srcp, dstp, zeros32, out, src_v, dst_v, rows_v, agg_sh, sems):
    c = lax.axis_index("c")
    s = lax.axis_index("s")
    wid = c * NS + s
    pltpu.sync_copy(srcp.at[wid], src_v)
    pltpu.sync_copy(dstp.at[wid], dst_v)
    pltpu.sync_copy(zeros32.at[pl.ds(s * RPT, RPT)], agg_sh.at[pl.ds(s * RPT, RPT)])
    plsc.subcore_barrier()

    pltpu.async_copy(h.at[src_v.at[0]], rows_v.at[0], sems.at[0])

    def body(j, carry):
        b = lax.rem(j, 2)
        nb = lax.rem(j + 1, 2)

        @pl.when(j + 1 < CH)
        def _():
            pltpu.async_copy(h.at[src_v.at[j + 1]], rows_v.at[nb], sems.at[nb])

        pltpu.make_async_copy(h.at[src_v.at[j]], rows_v.at[b], sems.at[b]).wait()
        pltpu.sync_copy(rows_v.at[b], agg_sh.at[dst_v.at[j]], add=True)
        return carry

    lax.fori_loop(0, CH, body, 0)
    plsc.subcore_barrier()
    pltpu.sync_copy(agg_sh.at[pl.ds(s * RPT, RPT)], out.at[c, pl.ds(s * RPT, RPT)])


_conv = functools.partial(
    pl.kernel,
    out_type=jax.ShapeDtypeStruct((NC, NP, H), jnp.float32),
    mesh=_mesh,
    compiler_params=pltpu.CompilerParams(use_tc_tiling_on_sc=False),
    scratch_types=[
        pltpu.VMEM((CH, B), jnp.int32),
        pltpu.VMEM((CH, B), jnp.int32),
        pltpu.VMEM((2, B, H), jnp.float32),
        pltpu.VMEM_SHARED((NP, H), jnp.float32),
        pltpu.SemaphoreType.DMA((2,)),
    ],
)(_conv_body)


_BLK = 1024
_GRID = NP // _BLK


def _norms(degp_ref):
    deg = degp_ref[0] + degp_ref[1]
    do_ = deg[:, 0:1]
    di = deg[:, 1:2]
    ns = jnp.where(do_ > 0, lax.rsqrt(jnp.maximum(do_, 1.0)), 0.0)
    nd = jnp.where(di > 0, lax.rsqrt(jnp.maximum(di, 1.0)), 0.0)
    return ns, nd


def _tcA_body(x_ref, w_ref, degp_ref, o_ref):
    ns, _ = _norms(degp_ref)
    o_ref[...] = jnp.dot(x_ref[...], w_ref[...],
                         preferred_element_type=jnp.float32) * ns


def _tcA(x_p, W1, degp):
    return pl.pallas_call(
        _tcA_body,
        grid=(_GRID,),
        in_specs=[
            pl.BlockSpec((_BLK, D), lambda i: (i, 0)),
            pl.BlockSpec((D, H), lambda i: (0, 0)),
            pl.BlockSpec((NC, _BLK, 8), lambda i: (0, i, 0)),
        ],
        out_specs=pl.BlockSpec((_BLK, H), lambda i: (i, 0)),
        out_shape=jax.ShapeDtypeStruct((NP, H), jnp.float32),
    )(x_p, W1, degp)


def _tcB_body(aggp_ref, degp_ref, b_ref, w_ref, o_ref):
    ns, nd = _norms(degp_ref)
    agg = aggp_ref[0] + aggp_ref[1]
    h = jax.nn.relu(agg * nd + b_ref[...])
    o_ref[...] = jnp.dot(h, w_ref[...], preferred_element_type=jnp.float32) * ns


def _tcB(aggp, degp, b1, W2):
    return pl.pallas_call(
        _tcB_body,
        grid=(_GRID,),
        in_specs=[
            pl.BlockSpec((NC, _BLK, H), lambda i: (0, i, 0)),
            pl.BlockSpec((NC, _BLK, 8), lambda i: (0, i, 0)),
            pl.BlockSpec((1, H), lambda i: (0, 0)),
            pl.BlockSpec((H, H), lambda i: (0, 0)),
        ],
        out_specs=pl.BlockSpec((_BLK, H), lambda i: (i, 0)),
        out_shape=jax.ShapeDtypeStruct((NP, H), jnp.float32),
    )(aggp, degp, b1.reshape(1, H), W2)


def _tcC_body(aggp_ref, degp_ref, b_ref, w_ref, bm_ref, o_ref):
    _, nd = _norms(degp_ref)
    agg = aggp_ref[0] + aggp_ref[1]
    h = jax.nn.relu(agg * nd + b_ref[...])
    o_ref[...] = jnp.dot(h, w_ref[...],
                         preferred_element_type=jnp.float32) + bm_ref[...]


def _tcC(aggp, degp, b2, Wm, bm):
    return pl.pallas_call(
        _tcC_body,
        grid=(_GRID,),
        in_specs=[
            pl.BlockSpec((NC, _BLK, H), lambda i: (0, i, 0)),
            pl.BlockSpec((NC, _BLK, 8), lambda i: (0, i, 0)),
            pl.BlockSpec((1, H), lambda i: (0, 0)),
            pl.BlockSpec((H, C), lambda i: (0, 0)),
            pl.BlockSpec((1, C), lambda i: (0, 0)),
        ],
        out_specs=pl.BlockSpec((_BLK, C), lambda i: (i, 0)),
        out_shape=jax.ShapeDtypeStruct((NP, C), jnp.float32),
    )(aggp, degp, b2.reshape(1, H), Wm, bm.reshape(1, C))


def kernel(x, edge_index, W1, b1, W2, b2, Wm, bm):
    src = edge_index[0].reshape(NW, EPW)
    dst = edge_index[1].reshape(NW, EPW)
    padblk = jnp.full((NW, EPWP - EPW), PAD, jnp.int32)
    srcp = jnp.concatenate([src, padblk], axis=1).reshape(NW, CH, B)
    dstp = jnp.concatenate([dst, padblk], axis=1).reshape(NW, CH, B)

    x_p = jnp.pad(x, ((0, NP - N), (0, 0)))
    ones2 = jnp.zeros((2, B, 8), jnp.float32).at[0, :, 0].set(1.0).at[1, :, 1].set(1.0)
    zeros8 = jnp.zeros((NP, 8), jnp.float32)
    zeros32 = jnp.zeros((NP, H), jnp.float32)

    degp = _deg(srcp, dstp, ones2, zeros8)
    h1a = _tcA(x_p, W1, degp)
    agg1 = _conv(h1a, srcp, dstp, zeros32)
    h2a = _tcB(agg1, degp, b1, W2)
    agg2 = _conv(h2a, srcp, dstp, zeros32)
    out = _tcC(agg2, degp, b2, Wm, bm)
    return out[:N]
